# baseline (speedup 1.0000x reference)
import functools

import jax
import jax.numpy as jnp
from jax.experimental import pallas as pl
from jax.experimental.pallas import tpu as pltpu

E = 8
TOP_K = 2
D_IN = 1024
D_OUT = 1024
D_PROJ = 256
N_TOK = 4096

BM = 1024
CH = 128
LANES = 128
D_CAT = E * D_PROJ

_NEG = -1e30


def _gelu_tanh(x):
    return 0.5 * x * (1.0 + jnp.tanh(jnp.sqrt(2.0 / jnp.pi) * (x + 0.044715 * x ** 3)))


def _moe_kernel(x_ref, wg_ref, bg_ref, w1_ref, b1_ref, w2_ref, b2_ref,
                out_ref):
    lane = jax.lax.broadcasted_iota(jnp.int32, (CH, LANES), 1)
    for c in range(BM // CH):
        rows = pl.ds(c * CH, CH)
        xb = x_ref[rows, :]

        logits = (jnp.dot(xb, wg_ref[...], preferred_element_type=jnp.float32)
                  + bg_ref[...]) * (1.0 / jnp.sqrt(jnp.float32(D_IN)))
        logits = jnp.where(lane < E, logits, _NEG)
        m1 = jnp.max(logits, axis=1, keepdims=True)
        p = jnp.exp(logits - m1)
        probs = p / jnp.sum(p, axis=1, keepdims=True)
        i1 = jnp.min(jnp.where(logits >= m1, lane, LANES), axis=1, keepdims=True)
        logits2 = jnp.where(lane == i1, _NEG, logits)
        m2 = jnp.max(logits2, axis=1, keepdims=True)
        i2 = jnp.min(jnp.where(logits2 >= m2, lane, LANES), axis=1, keepdims=True)
        wfull = probs * ((lane == i1) | (lane == i2)).astype(jnp.float32)

        xb16 = xb.astype(jnp.bfloat16)
        cols = []
        for g in range(E):
            hg = (jnp.dot(xb16, w1_ref[g].astype(jnp.bfloat16),
                          preferred_element_type=jnp.float32)
                  + b1_ref[:, g * D_PROJ:(g + 1) * D_PROJ])
            cols.append((_gelu_tanh(hg) * wfull[:, g:g + 1]).astype(jnp.bfloat16))
        g16 = jnp.concatenate(cols, axis=1)
        y = jnp.dot(g16, w2_ref[...].astype(jnp.bfloat16),
                    preferred_element_type=jnp.float32)
        y += jnp.dot(wfull.astype(jnp.bfloat16), b2_ref[...].astype(jnp.bfloat16),
                     preferred_element_type=jnp.float32)
        out_ref[rows, :] = y


@jax.jit
def kernel(x, Wg, bg, W1, b1, W2, b2):
    in_shape = x.shape
    xf = x.reshape(-1, D_IN)
    n = xf.shape[0]
    wg_pad = jnp.pad(Wg, ((0, 0), (0, LANES - E)))
    bg_pad = jnp.pad(bg, (0, LANES - E)).reshape(1, LANES)
    b1_cat = b1.reshape(1, D_CAT)
    w2_stack = W2.reshape(D_CAT, D_OUT)
    b2_pad = jnp.pad(b2, ((0, LANES - E), (0, 0)))
    grid = (n // BM,)
    y = pl.pallas_call(
        _moe_kernel,
        grid=grid,
        in_specs=[
            pl.BlockSpec((BM, D_IN), lambda i: (i, 0)),
            pl.BlockSpec((D_IN, LANES), lambda i: (0, 0)),
            pl.BlockSpec((1, LANES), lambda i: (0, 0)),
            pl.BlockSpec((E, D_IN, D_PROJ), lambda i: (0, 0, 0)),
            pl.BlockSpec((1, D_CAT), lambda i: (0, 0)),
            pl.BlockSpec((D_CAT, D_OUT), lambda i: (0, 0)),
            pl.BlockSpec((LANES, D_OUT), lambda i: (0, 0)),
        ],
        out_specs=pl.BlockSpec((BM, D_OUT), lambda i: (i, 0)),
        out_shape=jax.ShapeDtypeStruct((n, D_OUT), jnp.float32),
        compiler_params=pltpu.CompilerParams(
            dimension_semantics=("parallel",)),
    )(xf, wg_pad, bg_pad, W1, b1_cat, w2_stack, b2_pad)
    return y.reshape(in_shape[:-1] + (D_OUT,))


# speedup vs baseline: 1.0126x; 1.0126x over previous
import functools

import jax
import jax.numpy as jnp
from jax.experimental import pallas as pl
from jax.experimental.pallas import tpu as pltpu

E = 8
TOP_K = 2
D_IN = 1024
D_OUT = 1024
D_PROJ = 256
N_TOK = 4096

BM = 1024
CH = 512
LANES = 128
D_CAT = E * D_PROJ

_NEG = -1e30


def _gelu_tanh(x):
    return 0.5 * x * (1.0 + jnp.tanh(jnp.sqrt(2.0 / jnp.pi) * (x + 0.044715 * x ** 3)))


def _moe_kernel(x_ref, wg_ref, bg_ref, w1_ref, b1_ref, w2_ref, b2_ref,
                out_ref):
    lane = jax.lax.broadcasted_iota(jnp.int32, (CH, LANES), 1)
    for c in range(BM // CH):
        rows = pl.ds(c * CH, CH)
        xb = x_ref[rows, :]

        logits = (jnp.dot(xb, wg_ref[...], preferred_element_type=jnp.float32)
                  + bg_ref[...]) * (1.0 / jnp.sqrt(jnp.float32(D_IN)))
        logits = jnp.where(lane < E, logits, _NEG)
        m1 = jnp.max(logits, axis=1, keepdims=True)
        p = jnp.exp(logits - m1)
        probs = p / jnp.sum(p, axis=1, keepdims=True)
        i1 = jnp.min(jnp.where(logits >= m1, lane, LANES), axis=1, keepdims=True)
        logits2 = jnp.where(lane == i1, _NEG, logits)
        m2 = jnp.max(logits2, axis=1, keepdims=True)
        i2 = jnp.min(jnp.where(logits2 >= m2, lane, LANES), axis=1, keepdims=True)
        wfull = probs * ((lane == i1) | (lane == i2)).astype(jnp.float32)

        xb16 = xb.astype(jnp.bfloat16)
        cols = []
        for g in range(E):
            hg = (jnp.dot(xb16, w1_ref[g].astype(jnp.bfloat16),
                          preferred_element_type=jnp.float32)
                  + b1_ref[:, g * D_PROJ:(g + 1) * D_PROJ])
            cols.append((_gelu_tanh(hg) * wfull[:, g:g + 1]).astype(jnp.bfloat16))
        g16 = jnp.concatenate(cols, axis=1)
        y = jnp.dot(g16, w2_ref[...].astype(jnp.bfloat16),
                    preferred_element_type=jnp.float32)
        y += jnp.dot(wfull.astype(jnp.bfloat16), b2_ref[...].astype(jnp.bfloat16),
                     preferred_element_type=jnp.float32)
        out_ref[rows, :] = y


@jax.jit
def kernel(x, Wg, bg, W1, b1, W2, b2):
    in_shape = x.shape
    xf = x.reshape(-1, D_IN)
    n = xf.shape[0]
    wg_pad = jnp.pad(Wg, ((0, 0), (0, LANES - E)))
    bg_pad = jnp.pad(bg, (0, LANES - E)).reshape(1, LANES)
    b1_cat = b1.reshape(1, D_CAT)
    w2_stack = W2.reshape(D_CAT, D_OUT)
    b2_pad = jnp.pad(b2, ((0, LANES - E), (0, 0)))
    grid = (n // BM,)
    y = pl.pallas_call(
        _moe_kernel,
        grid=grid,
        in_specs=[
            pl.BlockSpec((BM, D_IN), lambda i: (i, 0)),
            pl.BlockSpec((D_IN, LANES), lambda i: (0, 0)),
            pl.BlockSpec((1, LANES), lambda i: (0, 0)),
            pl.BlockSpec((E, D_IN, D_PROJ), lambda i: (0, 0, 0)),
            pl.BlockSpec((1, D_CAT), lambda i: (0, 0)),
            pl.BlockSpec((D_CAT, D_OUT), lambda i: (0, 0)),
            pl.BlockSpec((LANES, D_OUT), lambda i: (0, 0)),
        ],
        out_specs=pl.BlockSpec((BM, D_OUT), lambda i: (i, 0)),
        out_shape=jax.ShapeDtypeStruct((n, D_OUT), jnp.float32),
        compiler_params=pltpu.CompilerParams(
            dimension_semantics=("parallel",)),
    )(xf, wg_pad, bg_pad, W1, b1_cat, w2_stack, b2_pad)
    return y.reshape(in_shape[:-1] + (D_OUT,))


# speedup vs baseline: 1.1499x; 1.1355x over previous
import functools

import jax
import jax.numpy as jnp
from jax.experimental import pallas as pl
from jax.experimental.pallas import tpu as pltpu

E = 8
TOP_K = 2
D_IN = 1024
D_OUT = 1024
D_PROJ = 256
N_TOK = 4096

BM = 2048
CH = 256
LANES = 128
D_CAT = E * D_PROJ

_NEG = -1e30


def _gelu_tanh(x):
    return 0.5 * x * (1.0 + jnp.tanh(jnp.sqrt(2.0 / jnp.pi) * (x + 0.044715 * x ** 3)))


def _moe_kernel(x_ref, wg_ref, bg_ref, w1_ref, b1_ref, w2_ref, b2_ref,
                out_ref):
    lane = jax.lax.broadcasted_iota(jnp.int32, (CH, LANES), 1)
    for c in range(BM // CH):
        rows = pl.ds(c * CH, CH)
        xb = x_ref[rows, :]

        logits = (jnp.dot(xb, wg_ref[...], preferred_element_type=jnp.float32)
                  + bg_ref[...]) * (1.0 / jnp.sqrt(jnp.float32(D_IN)))
        logits = jnp.where(lane < E, logits, _NEG)
        m1 = jnp.max(logits, axis=1, keepdims=True)
        p = jnp.exp(logits - m1)
        probs = p / jnp.sum(p, axis=1, keepdims=True)
        i1 = jnp.min(jnp.where(logits >= m1, lane, LANES), axis=1, keepdims=True)
        logits2 = jnp.where(lane == i1, _NEG, logits)
        m2 = jnp.max(logits2, axis=1, keepdims=True)
        i2 = jnp.min(jnp.where(logits2 >= m2, lane, LANES), axis=1, keepdims=True)
        wfull = probs * ((lane == i1) | (lane == i2)).astype(jnp.float32)

        xb16 = xb.astype(jnp.bfloat16)
        cols = []
        for g in range(E):
            hg = (jnp.dot(xb16, w1_ref[g].astype(jnp.bfloat16),
                          preferred_element_type=jnp.float32)
                  + b1_ref[:, g * D_PROJ:(g + 1) * D_PROJ])
            cols.append((_gelu_tanh(hg) * wfull[:, g:g + 1]).astype(jnp.bfloat16))
        g16 = jnp.concatenate(cols, axis=1)
        y = jnp.dot(g16, w2_ref[...].astype(jnp.bfloat16),
                    preferred_element_type=jnp.float32)
        y += jnp.dot(wfull.astype(jnp.bfloat16), b2_ref[...].astype(jnp.bfloat16),
                     preferred_element_type=jnp.float32)
        out_ref[rows, :] = y


@jax.jit
def kernel(x, Wg, bg, W1, b1, W2, b2):
    in_shape = x.shape
    xf = x.reshape(-1, D_IN)
    n = xf.shape[0]
    wg_pad = jnp.pad(Wg, ((0, 0), (0, LANES - E)))
    bg_pad = jnp.pad(bg, (0, LANES - E)).reshape(1, LANES)
    b1_cat = b1.reshape(1, D_CAT)
    w2_stack = W2.reshape(D_CAT, D_OUT)
    b2_pad = jnp.pad(b2, ((0, LANES - E), (0, 0)))
    grid = (n // BM,)
    y = pl.pallas_call(
        _moe_kernel,
        grid=grid,
        in_specs=[
            pl.BlockSpec((BM, D_IN), lambda i: (i, 0)),
            pl.BlockSpec((D_IN, LANES), lambda i: (0, 0)),
            pl.BlockSpec((1, LANES), lambda i: (0, 0)),
            pl.BlockSpec((E, D_IN, D_PROJ), lambda i: (0, 0, 0)),
            pl.BlockSpec((1, D_CAT), lambda i: (0, 0)),
            pl.BlockSpec((D_CAT, D_OUT), lambda i: (0, 0)),
            pl.BlockSpec((LANES, D_OUT), lambda i: (0, 0)),
        ],
        out_specs=pl.BlockSpec((BM, D_OUT), lambda i: (i, 0)),
        out_shape=jax.ShapeDtypeStruct((n, D_OUT), jnp.float32),
        compiler_params=pltpu.CompilerParams(
            dimension_semantics=("parallel",)),
    )(xf, wg_pad, bg_pad, W1, b1_cat, w2_stack, b2_pad)
    return y.reshape(in_shape[:-1] + (D_OUT,))
